# baseline (speedup 1.0000x reference)
import functools

import jax
import jax.numpy as jnp
from jax import lax
from jax.experimental import pallas as pl
from jax.experimental.pallas import tpu as pltpu
from jax.experimental.pallas import tpu_sc as plsc

_DIM = 64
_SCALE = 8.0
_CHUNK = 128
_NW = 32
_B0 = 4096
_B1 = 200
_B = _B0 * _B1
_BLOCKS = _B // _CHUNK
_BLK_PER_W = _BLOCKS // _NW
_NBUF = 4
_B0_BLKS = _B0 // _CHUNK


def _body(xt_hbm, table_hbm, out_hbm, idx_v, grows, trows, gsem, ssem):
    cid = lax.axis_index("c")
    sid = lax.axis_index("s")
    wid = sid * 2 + cid
    base_blk = wid * _BLK_PER_W

    pltpu.sync_copy(xt_hbm.at[pl.ds(base_blk, _BLK_PER_W)], idx_v)

    def gather_start(j, b):
        pltpu.async_copy(table_hbm.at[idx_v.at[j]], grows.at[b], gsem.at[b])

    def gather_wait(j, b):
        pltpu.make_async_copy(table_hbm.at[idx_v.at[j]], grows.at[b],
                              gsem.at[b]).wait()

    def out_slice(j):
        c = base_blk + j
        b1 = lax.shift_right_logical(c, 5)
        b0b = c & (_B0_BLKS - 1)
        return out_hbm.at[b1, :, b0b, :, :]

    def scatter_start(j, b):
        pltpu.async_copy(trows.at[b], out_slice(j), ssem.at[b])

    def scatter_wait(j, b):
        pltpu.make_async_copy(trows.at[b], out_slice(j), ssem.at[b]).wait()

    for b in range(_NBUF):
        gather_start(b, b)

    iota16 = lax.iota(jnp.int32, 16)

    def outer(jo, carry):
        for b in range(_NBUF):
            j = jo * _NBUF + b
            gather_wait(j, b)

            @pl.when(jo > 0)
            def _():
                scatter_wait(j - _NBUF, b)

            @plsc.parallel_loop(0, _CHUNK // 16)
            def _(k):
                rows = k * 16 + iota16
                for m in range(_DIM // 16):
                    for dj in range(16):
                        dsel = m * 16 + ((iota16 + dj) & 15)
                        v = plsc.load_gather(grows.at[b], [rows, dsel])
                        plsc.store_scatter(
                            trows.at[b],
                            [lax.shift_right_logical(dsel, 3), dsel & 7, rows],
                            v * _SCALE)

            scatter_start(j, b)

            @pl.when(j + _NBUF < _BLK_PER_W)
            def _():
                gather_start(j + _NBUF, b)
        return carry

    lax.fori_loop(0, _BLK_PER_W // _NBUF, outer, 0)

    for b in range(_NBUF):
        scatter_wait(_BLK_PER_W - _NBUF + b, b)


@jax.jit
def _embed(xt2, table):
    mesh = plsc.VectorSubcoreMesh(core_axis_name="c", subcore_axis_name="s")
    kfn = pl.kernel(
        _body,
        out_type=jax.ShapeDtypeStruct((_B1, _DIM // 8, _B0_BLKS, 8, _CHUNK),
                                      jnp.float32),
        mesh=mesh,
        scratch_types=[
            pltpu.VMEM((_BLK_PER_W, _CHUNK), jnp.int32),
            pltpu.VMEM((_NBUF, _CHUNK, _DIM), jnp.float32),
            pltpu.VMEM((_NBUF, _DIM // 8, 8, _CHUNK), jnp.float32),
            pltpu.SemaphoreType.DMA((_NBUF,)),
            pltpu.SemaphoreType.DMA((_NBUF,)),
        ],
        compiler_params=pltpu.CompilerParams(use_tc_tiling_on_sc=False,
                                             needs_layout_passes=False),
    )
    return kfn(xt2, table)


def kernel(x, table):
    xt2 = x.T.reshape(_BLOCKS, _CHUNK)
    out5 = _embed(xt2, table)
    t = jnp.transpose(out5, (2, 4, 0, 1, 3))
    return t.reshape(_B0, _B1, _DIM)


# speedup vs baseline: 1.1256x; 1.1256x over previous
import functools

import jax
import jax.numpy as jnp
from jax import lax
from jax.experimental import pallas as pl
from jax.experimental.pallas import tpu as pltpu
from jax.experimental.pallas import tpu_sc as plsc

_DIM = 64
_SCALE = 8.0
_CHUNK = 128
_NW = 32
_B0 = 4096
_B1 = 200
_B = _B0 * _B1
_BLOCKS = _B // _CHUNK
_BLK_PER_W = _BLOCKS // _NW
_NBUF = 4
_B0_BLKS = _B0 // _CHUNK
_CHW = _CHUNK * _DIM
_NTRIP = _CHW // 16


def _body(xt_hbm, table_hbm, out_hbm, idx_v, g_ring, t_ring, sidx, didx,
          gsem, ssem):
    cid = lax.axis_index("c")
    sid = lax.axis_index("s")
    wid = sid * 2 + cid
    base_blk = wid * _BLK_PER_W

    pltpu.sync_copy(xt_hbm.at[pl.ds(base_blk, _BLK_PER_W)], idx_v)

    iota16 = lax.iota(jnp.int32, 16)

    def mktab(t, carry):
        rows = (lax.shift_right_logical(t, 6) * 16) + iota16
        dsel = (lax.shift_right_logical(t, 4) & 3) * 16 + ((iota16 + (t & 15)) & 15)
        sidx[t, :] = dsel
        didx[t, :] = dsel * _CHUNK + rows
        return carry

    lax.fori_loop(0, _NTRIP, mktab, 0)

    def gather_start(j, b):
        pltpu.async_copy(table_hbm.at[idx_v.at[j]], g_ring.at[b], gsem.at[b])

    def gather_wait(j, b):
        pltpu.make_async_copy(table_hbm.at[idx_v.at[j]], g_ring.at[b],
                              gsem.at[b]).wait()

    def out_base(j):
        c = base_blk + j
        b1 = lax.shift_right_logical(c, 5)
        b0b = c & (_B0_BLKS - 1)
        return b1 * (_DIM * _B0) + b0b * (8 * _CHUNK)

    def scatter_start(j, b):
        ob = out_base(j)
        for dt in range(8):
            pltpu.async_copy(
                t_ring.at[b, pl.ds(dt * 1024, 1024)],
                out_hbm.at[pl.ds(ob + dt * (_B0_BLKS * 8 * _CHUNK), 1024)],
                ssem.at[b])

    def scatter_wait(j, b):
        pltpu.make_async_copy(t_ring.at[b],
                              out_hbm.at[pl.ds(out_base(j), _CHW)],
                              ssem.at[b]).wait()

    for b in range(_NBUF):
        gather_start(b, b)

    def outer(jo, carry):
        for b in range(_NBUF):
            j = jo * _NBUF + b
            gather_wait(j, b)

            @pl.when(jo > 0)
            def _():
                scatter_wait(j - _NBUF, b)

            @plsc.parallel_loop(0, _NTRIP // 32)
            def _(k2):
                rows = lax.shift_right_logical(k2, 1) * 16 + iota16
                for u in range(32):
                    t = k2 * 32 + u
                    v = plsc.load_gather(g_ring.at[b], [rows, sidx[t, :]])
                    plsc.store_scatter(t_ring.at[b], [didx[t, :]], v * _SCALE)

            scatter_start(j, b)

            @pl.when(j + _NBUF < _BLK_PER_W)
            def _():
                gather_start(j + _NBUF, b)
        return carry

    lax.fori_loop(0, _BLK_PER_W // _NBUF, outer, 0)

    for b in range(_NBUF):
        scatter_wait(_BLK_PER_W - _NBUF + b, b)


@jax.jit
def _embed(xt2, table):
    mesh = plsc.VectorSubcoreMesh(core_axis_name="c", subcore_axis_name="s")
    kfn = pl.kernel(
        _body,
        out_type=jax.ShapeDtypeStruct((_B * _DIM,), jnp.float32),
        mesh=mesh,
        scratch_types=[
            pltpu.VMEM((_BLK_PER_W, _CHUNK), jnp.int32),
            pltpu.VMEM((_NBUF, _CHUNK, _DIM), jnp.float32),
            pltpu.VMEM((_NBUF, _CHW), jnp.float32),
            pltpu.VMEM((_NTRIP, 16), jnp.int32),
            pltpu.VMEM((_NTRIP, 16), jnp.int32),
            pltpu.SemaphoreType.DMA((_NBUF,)),
            pltpu.SemaphoreType.DMA((_NBUF,)),
        ],
        compiler_params=pltpu.CompilerParams(use_tc_tiling_on_sc=False,
                                             needs_layout_passes=False),
    )
    return kfn(xt2, table)


def kernel(x, table):
    xt2 = x.T.reshape(_BLOCKS, _CHUNK)
    flat = _embed(xt2, table)
    out5 = flat.reshape(_B1, _DIM // 8, _B0_BLKS, 8, _CHUNK)
    t = jnp.transpose(out5, (2, 4, 0, 1, 3))
    return t.reshape(_B0, _B1, _DIM)


# speedup vs baseline: 1.3104x; 1.1642x over previous
import functools

import jax
import jax.numpy as jnp
from jax import lax
from jax.experimental import pallas as pl
from jax.experimental.pallas import tpu as pltpu
from jax.experimental.pallas import tpu_sc as plsc

_DIM = 64
_SCALE = 8.0
_CHUNK = 128
_NW = 32
_B0 = 4096
_B1 = 200
_B = _B0 * _B1
_BLOCKS = _B // _CHUNK
_BLK_PER_W = _BLOCKS // _NW
_NBUF = 4
_B0_BLKS = _B0 // _CHUNK
_CHW = _CHUNK * _DIM
_NTRIP = _CHW // 16


def _body(xt_hbm, table_hbm, out_hbm, idx_v, g_ring, t_ring, gsem, ssem):
    cid = lax.axis_index("c")
    sid = lax.axis_index("s")
    wid = sid * 2 + cid
    base_blk = wid * _BLK_PER_W

    pltpu.sync_copy(xt_hbm.at[pl.ds(base_blk, _BLK_PER_W)], idx_v)

    iota16 = lax.iota(jnp.int32, 16)

    def gather_start(j, b):
        pltpu.async_copy(table_hbm.at[idx_v.at[j]], g_ring.at[b], gsem.at[b])

    def gather_wait(j, b):
        pltpu.make_async_copy(table_hbm.at[idx_v.at[j]], g_ring.at[b],
                              gsem.at[b]).wait()

    def out_base(j):
        c = base_blk + j
        b1 = lax.shift_right_logical(c, 5)
        b0b = c & (_B0_BLKS - 1)
        return b1 * (_DIM * _B0) + b0b * (8 * _CHUNK)

    def scatter_start(j, b):
        ob = out_base(j)
        for dt in range(8):
            pltpu.async_copy(
                t_ring.at[b, pl.ds(dt * 1024, 1024)],
                out_hbm.at[pl.ds(ob + dt * (_B0_BLKS * 8 * _CHUNK), 1024)],
                ssem.at[b])

    def scatter_wait(j, b):
        pltpu.make_async_copy(t_ring.at[b],
                              out_hbm.at[pl.ds(out_base(j), _CHW)],
                              ssem.at[b]).wait()

    for b in range(_NBUF):
        gather_start(b, b)

    def outer(jo, carry):
        for b in range(_NBUF):
            j = jo * _NBUF + b
            gather_wait(j, b)

            @pl.when(jo > 0)
            def _():
                scatter_wait(j - _NBUF, b)

            @plsc.parallel_loop(0, 16)
            def _(dj):
                rot = (iota16 + dj) & 15
                for k in range(_CHUNK // 16):
                    rows = k * 16 + iota16
                    for m in range(_DIM // 16):
                        dsel = m * 16 + rot
                        v = plsc.load_gather(g_ring.at[b], [rows, dsel])
                        plsc.store_scatter(t_ring.at[b],
                                           [dsel * _CHUNK + rows], v * _SCALE)

            scatter_start(j, b)

            @pl.when(j + _NBUF < _BLK_PER_W)
            def _():
                gather_start(j + _NBUF, b)
        return carry

    lax.fori_loop(0, _BLK_PER_W // _NBUF, outer, 0)

    for b in range(_NBUF):
        scatter_wait(_BLK_PER_W - _NBUF + b, b)


@jax.jit
def _embed(xt2, table):
    mesh = plsc.VectorSubcoreMesh(core_axis_name="c", subcore_axis_name="s")
    kfn = pl.kernel(
        _body,
        out_type=jax.ShapeDtypeStruct((_B * _DIM,), jnp.float32),
        mesh=mesh,
        scratch_types=[
            pltpu.VMEM((_BLK_PER_W, _CHUNK), jnp.int32),
            pltpu.VMEM((_NBUF, _CHUNK, _DIM), jnp.float32),
            pltpu.VMEM((_NBUF, _CHW), jnp.float32),
            pltpu.SemaphoreType.DMA((_NBUF,)),
            pltpu.SemaphoreType.DMA((_NBUF,)),
        ],
        compiler_params=pltpu.CompilerParams(use_tc_tiling_on_sc=False,
                                             needs_layout_passes=False),
    )
    return kfn(xt2, table)


def kernel(x, table):
    xt2 = x.T.reshape(_BLOCKS, _CHUNK)
    flat = _embed(xt2, table)
    out5 = flat.reshape(_B1, _DIM // 8, _B0_BLKS, 8, _CHUNK)
    t = jnp.transpose(out5, (2, 4, 0, 1, 3))
    return t.reshape(_B0, _B1, _DIM)
